# baseline (speedup 1.0000x reference)
import functools

import jax
import jax.numpy as jnp
from jax import lax
from jax.experimental import pallas as pl
from jax.experimental.pallas import tpu as pltpu
from jax.experimental.pallas import tpu_sc as plsc

B, NF, NT, V, D, L = 4096, 10, 26, 100000, 32, 20
DENSE_OUT, OVER_OUT = 32, 16

_NC = 2
_NS = 16
_NW = _NC * _NS

_BPW = B // _NW
_CP = 16
_NCB = _BPW // _CP
_NU = NT * _NCB
_SL = 64
_RS = _CP * L // _SL
_GB = V * D // 128


@functools.cache
def _get_sc_pool():
    mesh = plsc.VectorSubcoreMesh(core_axis_name="c", subcore_axis_name="s")
    return functools.partial(
        pl.kernel,
        mesh=mesh,
        compiler_params=pltpu.CompilerParams(use_tc_tiling_on_sc=False),
        out_type=jax.ShapeDtypeStruct((NT, B * D // 128, 128), jnp.float32),
        scratch_types=[
            pltpu.VMEM((2, _CP * L + 16), jnp.int32),
            pltpu.VMEM((2, _RS, _SL), jnp.int32),
            pltpu.VMEM((2, _RS, _SL, 128), jnp.float32),
            pltpu.VMEM((2, _CP * D // 128, 128), jnp.float32),
            pltpu.SemaphoreType.DMA,
            pltpu.SemaphoreType.DMA,
        ],
    )(_sc_pool_body)


def _sc_pool_body(tables_hbm, idx_hbm, out_hbm, idx_v, sidx_v, rows_v, out_v,
                  sem0, sem1):
    wid = lax.axis_index("s") * _NC + lax.axis_index("c")
    b0 = wid * _BPW
    sems = (sem0, sem1)

    def start(slot, u, sem):
        t = u >> 3
        c = u & 7
        pltpu.sync_copy(idx_hbm.at[t, wid, c],
                        idx_v.at[slot, pl.ds(0, _CP * L)])
        for v in range(_CP * L // 16):
            j = v // (_SL // 16)
            k0 = (v % (_SL // 16)) * 16
            sidx_v[slot, j, pl.ds(k0, 16)] = (
                idx_v[slot, pl.ds(v * 16, 16)] >> 2)
        for j in range(_RS):
            pltpu.async_copy(tables_hbm.at[t].at[sidx_v.at[slot, j]],
                             rows_v.at[slot, j], sem)

    def drain(slot, u, sem):
        t = u >> 3
        for j in range(_RS):
            pltpu.make_async_copy(tables_hbm.at[t].at[sidx_v.at[slot, j]],
                                  rows_v.at[slot, j], sem).wait()

    def compute(slot):
        def pair_body(p, carry):
            base = p * L
            acc_a = None
            acc_b = None
            for l in range(L):
                r = base + l
                j = r >> 6
                k = r & 63
                o = (idx_v[slot, pl.ds(r, 16)][0] & 3) * 32
                va = rows_v[slot, j, k, pl.ds(o, 16)]
                vb = rows_v[slot, j, k, pl.ds(o + 16, 16)]
                acc_a = va if acc_a is None else acc_a + va
                acc_b = vb if acc_b is None else acc_b + vb
            q = (p & 3) * 32
            out_v[slot, p >> 2, pl.ds(q, 16)] = acc_a
            out_v[slot, p >> 2, pl.ds(q + 16, 16)] = acc_b
            return carry

        lax.fori_loop(0, _CP, pair_body, 0)

    start(0, 0, sems[0])
    start(1, 1, sems[1])

    def loop_body(u2, carry):
        for slot in range(2):
            u = u2 * 2 + slot
            drain(slot, u, sems[slot])
            compute(slot)
            pltpu.sync_copy(
                out_v.at[slot],
                out_hbm.at[u >> 3, pl.ds((b0 + (u & 7) * _CP) * D // 128,
                                         _CP * D // 128)])
            nxt = u + 2

            @pl.when(nxt < _NU)
            def _():
                start(slot, nxt, sems[slot])
        return carry

    lax.fori_loop(0, _NU // 2, loop_body, 0)


_BM = 512


def _head_body(ff, dw, db, pooled, owd, ows, ob, o):
    dense = jnp.maximum(
        jnp.dot(ff[:], dw[:], preferred_element_type=jnp.float32) + db[:], 0.0)
    acc = jnp.dot(dense, owd[:], preferred_element_type=jnp.float32) + ob[:]
    for t in range(NT):
        acc = acc + jnp.dot(pooled[t], ows[t],
                            preferred_element_type=jnp.float32)
    o[:] = acc


_tc_head = pl.pallas_call(
    _head_body,
    grid=(B // _BM,),
    in_specs=[
        pl.BlockSpec((_BM, NF), lambda i: (i, 0)),
        pl.BlockSpec((NF, DENSE_OUT), lambda i: (0, 0)),
        pl.BlockSpec((1, DENSE_OUT), lambda i: (0, 0)),
        pl.BlockSpec((NT, _BM, D), lambda i: (0, i, 0)),
        pl.BlockSpec((DENSE_OUT, OVER_OUT), lambda i: (0, 0)),
        pl.BlockSpec((NT, D, OVER_OUT), lambda i: (0, 0, 0)),
        pl.BlockSpec((1, OVER_OUT), lambda i: (0, 0)),
    ],
    out_specs=pl.BlockSpec((_BM, OVER_OUT), lambda i: (i, 0)),
    out_shape=jax.ShapeDtypeStruct((B, OVER_OUT), jnp.float32),
)


def kernel(float_features, indices, tables, dense_w, dense_b, over_w, over_b):
    tview = tables.reshape(NT, _GB, 128)
    idx_t = jnp.transpose(indices.astype(jnp.int32), (1, 0, 2))
    idx_chunks = idx_t.reshape(NT, _NW, _NCB, _CP * L)
    pooled = _get_sc_pool()(tview, idx_chunks)
    pooled3 = pooled.reshape(NT, B, D)
    ows = over_w[DENSE_OUT:].reshape(NT, D, OVER_OUT)
    out = _tc_head(float_features, dense_w, dense_b.reshape(1, DENSE_OUT),
                   pooled3, over_w[:DENSE_OUT], ows,
                   over_b.reshape(1, OVER_OUT))
    return out


# speedup vs baseline: 1.8248x; 1.8248x over previous
import functools

import jax
import jax.numpy as jnp
from jax import lax
from jax.experimental import pallas as pl
from jax.experimental.pallas import tpu as pltpu
from jax.experimental.pallas import tpu_sc as plsc

B, NF, NT, V, D, L = 4096, 10, 26, 100000, 32, 20
DENSE_OUT, OVER_OUT = 32, 16

_NC = 2
_NS = 16
_NW = _NC * _NS

_PAIRS = B * NT
_PPW = _PAIRS // _NW
_CP = 64
_NCH = _PPW // _CP
_SL = 128
_RS = _CP * L // _SL
_GCH = _PAIRS // _CP

@functools.cache
def _get_sc_pool():
    mesh = plsc.VectorSubcoreMesh(core_axis_name="c", subcore_axis_name="s")
    return functools.partial(
        pl.kernel,
        mesh=mesh,
        compiler_params=pltpu.CompilerParams(use_tc_tiling_on_sc=False),
        out_type=jax.ShapeDtypeStruct((_PAIRS, D), jnp.float32),
        scratch_types=[
            pltpu.VMEM((2, _RS, _SL), jnp.int32),
            pltpu.VMEM((2, _RS, _SL, D), jnp.float32),
            pltpu.VMEM((2, _CP, D), jnp.float32),
            pltpu.SemaphoreType.DMA,
            pltpu.SemaphoreType.DMA,
        ],
    )(_sc_pool_body)


def _sc_pool_body(tables_hbm, idx_hbm, out_hbm, idx_v, rows_v, out_v, sem0, sem1):
    wid = lax.axis_index("s") * _NC + lax.axis_index("c")
    sems = (sem0, sem1)

    def start(slot, ci, sem):
        pltpu.sync_copy(idx_hbm.at[ci], idx_v.at[slot])
        for j in range(_RS):
            pltpu.async_copy(tables_hbm.at[idx_v.at[slot, j]],
                             rows_v.at[slot, j], sem)

    def drain(slot, sem):
        for j in range(_RS):
            pltpu.make_async_copy(tables_hbm.at[idx_v.at[slot, j]],
                                  rows_v.at[slot, j], sem).wait()

    def compute(slot):
        def pair_body(p, carry):
            base = p * L
            acc_a = rows_v[slot, base >> 7, base & 127, pl.ds(0, 16)]
            acc_b = rows_v[slot, base >> 7, base & 127, pl.ds(16, 16)]
            for l in range(1, L):
                r = base + l
                j = r >> 7
                k = r & 127
                acc_a = acc_a + rows_v[slot, j, k, pl.ds(0, 16)]
                acc_b = acc_b + rows_v[slot, j, k, pl.ds(16, 16)]
            out_v[slot, p, pl.ds(0, 16)] = acc_a
            out_v[slot, p, pl.ds(16, 16)] = acc_b
            return carry

        lax.fori_loop(0, _CP, pair_body, 0)

    chunk0 = wid * _NCH
    start(0, chunk0, sems[0])
    start(1, chunk0 + 1, sems[1])

    def loop_body(c2, carry):
        for slot in range(2):
            ci = c2 * 2 + slot
            drain(slot, sems[slot])
            compute(slot)
            pltpu.sync_copy(
                out_v.at[slot],
                out_hbm.at[pl.ds((chunk0 + ci) * _CP, _CP)])
            nxt = ci + 2

            @pl.when(nxt < _NCH)
            def _():
                start(slot, chunk0 + nxt, sems[slot])
        return carry

    lax.fori_loop(0, _NCH // 2, loop_body, 0)


_TV = V


_Q = _TV // 4


def _lin_body(tin, o):
    x = tin[0]
    for a in range(4):
        o[:, a * D:(a + 1) * D] = jnp.transpose(x[:, a * _Q:(a + 1) * _Q])


_linearize = pl.pallas_call(
    _lin_body,
    grid=(NT,),
    compiler_params=pltpu.CompilerParams(vmem_limit_bytes=65011712),
    in_specs=[pl.BlockSpec((1, D, _TV), lambda t: (t, 0, 0))],
    out_specs=pl.BlockSpec((_Q, 128), lambda t: (t, 0)),
    out_shape=jax.ShapeDtypeStruct((NT * _Q, 128), jnp.float32),
)


_BM = 512


def _head_body(ff, dw, db, pooled, owd, ows, ob, o):
    dense = jnp.maximum(
        jnp.dot(ff[:], dw[:], preferred_element_type=jnp.float32) + db[:], 0.0)
    o[:] = (jnp.dot(dense, owd[:], preferred_element_type=jnp.float32)
            + jnp.dot(pooled[:], ows[:], preferred_element_type=jnp.float32)
            + ob[:])


_tc_head = pl.pallas_call(
    _head_body,
    grid=(B // _BM,),
    in_specs=[
        pl.BlockSpec((_BM, NF), lambda i: (i, 0)),
        pl.BlockSpec((NF, DENSE_OUT), lambda i: (0, 0)),
        pl.BlockSpec((1, DENSE_OUT), lambda i: (0, 0)),
        pl.BlockSpec((_BM, NT * D), lambda i: (i, 0)),
        pl.BlockSpec((DENSE_OUT, OVER_OUT), lambda i: (0, 0)),
        pl.BlockSpec((NT * D, OVER_OUT), lambda i: (0, 0)),
        pl.BlockSpec((1, OVER_OUT), lambda i: (0, 0)),
    ],
    out_specs=pl.BlockSpec((_BM, OVER_OUT), lambda i: (i, 0)),
    out_shape=jax.ShapeDtypeStruct((B, OVER_OUT), jnp.float32),
)


def kernel(float_features, indices, tables, dense_w, dense_b, over_w, over_b):
    tables_t = jnp.transpose(tables, (0, 2, 1))
    tables2d = _linearize(tables_t).reshape(NT * V, D)
    idx32 = indices.astype(jnp.int32)
    rho = (idx32 % _Q) * 4 + idx32 // _Q
    flat_idx = rho + (jnp.arange(NT, dtype=jnp.int32) * V)[None, :, None]
    idx_chunks = flat_idx.reshape(_GCH, _RS, _SL)
    pooled = _get_sc_pool()(tables2d, idx_chunks)
    pooled2 = pooled.reshape(B, NT * D)
    out = _tc_head(float_features, dense_w, dense_b.reshape(1, DENSE_OUT),
                   pooled2, over_w[:DENSE_OUT], over_w[DENSE_OUT:],
                   over_b.reshape(1, OVER_OUT))
    return out
